# baseline (device time: 55754 ns/iter reference)
import jax
import jax.numpy as jnp
from jax import lax
from jax.experimental import pallas as pl
from jax.experimental.pallas import tpu as pltpu

N_DEV = 16
N_TOK = 2048
D_IN = 512
D_OUT = 1024
N_EXP = 64
E_LOC = N_EXP // N_DEV
CAP = 25
SLOT_PER_EXP = 32
SLOTS = E_LOC * SLOT_PER_EXP
NQ = 4
QTR = SLOTS // NQ
CW_ROUNDS = 8
CCW_ROUNDS = 7


def kernel(x, router_W, route_idx, expert_W):
    del router_W

    def body(x_ref, idx_ref, w_ref, out_ref, comm_ref,
             cw_send_sems, cw_recv_sems, ccw_send_sems, ccw_recv_sems):
        my = lax.axis_index("i")
        left = lax.rem(my + N_DEV - 1, N_DEV)
        right = lax.rem(my + 1, N_DEV)

        barrier_sem = pltpu.get_barrier_semaphore()
        pl.semaphore_signal(barrier_sem, inc=1, device_id=(left,),
                            device_id_type=pl.DeviceIdType.MESH)
        pl.semaphore_signal(barrier_sem, inc=1, device_id=(right,),
                            device_id_type=pl.DeviceIdType.MESH)
        pl.semaphore_wait(barrier_sem, 2)

        BLK = 128
        NBLK = N_TOK // BLK
        FLAT = NBLK * N_EXP
        idx = idx_ref[:]
        idx2d = jnp.concatenate(
            [idx[b * BLK:(b + 1) * BLK] for b in range(NBLK)], axis=1)
        sel_io_r = lax.broadcasted_iota(jnp.int32, (NBLK, FLAT), 0)
        sel_io_c = lax.broadcasted_iota(jnp.int32, (NBLK, FLAT), 1)
        sel = (sel_io_c // N_EXP == sel_io_r).astype(jnp.bfloat16)
        idx_exp = jnp.dot(idx2d.astype(jnp.bfloat16), sel,
                          preferred_element_type=jnp.float32)
        e_io = lax.rem(
            lax.broadcasted_iota(jnp.int32, (BLK, FLAT), 1), N_EXP)
        onehot2 = (idx_exp == e_io.astype(jnp.float32)
                   ).astype(jnp.bfloat16)

        blk_sum = jnp.sum(onehot2, axis=0, keepdims=True).astype(jnp.bfloat16)
        blk16 = jnp.concatenate(
            [blk_sum[:, b * N_EXP:(b + 1) * N_EXP] for b in range(NBLK)],
            axis=0)
        r16 = lax.broadcasted_iota(jnp.int32, (NBLK, NBLK), 0)
        c16 = lax.broadcasted_iota(jnp.int32, (NBLK, NBLK), 1)
        l16 = (c16 < r16).astype(jnp.bfloat16)
        offs16 = jnp.dot(l16, blk16,
                         preferred_element_type=jnp.float32)
        offs_row = jnp.concatenate(
            [offs16[b:b + 1, :] for b in range(NBLK)], axis=1)

        r128 = lax.broadcasted_iota(jnp.int32, (BLK, BLK), 0)
        c128 = lax.broadcasted_iota(jnp.int32, (BLK, BLK), 1)
        l128 = (c128 < r128).astype(jnp.bfloat16)
        within2 = jax.lax.dot_general(
            l128, onehot2, (((1,), (0,)), ((), ())),
            preferred_element_type=jnp.float32)
        tot2 = ((within2 + offs_row) * onehot2.astype(jnp.float32)
                ).astype(jnp.bfloat16)
        grp = ((lax.broadcasted_iota(jnp.int32, (FLAT, NBLK), 0) // N_EXP)
               == lax.broadcasted_iota(jnp.int32, (FLAT, NBLK), 1)
               ).astype(jnp.bfloat16)
        pos2 = jnp.dot(tot2, grp,
                       preferred_element_type=jnp.float32)
        pos = jnp.concatenate(
            [pos2[:, b:b + 1] for b in range(NBLK)], axis=0
        ).astype(jnp.int32)
        keep = pos < CAP

        rel = lax.rem(idx // E_LOC - my + N_DEV, N_DEV)
        slot_rel = jnp.where(
            keep, rel * SLOTS + lax.rem(idx, E_LOC) * SLOT_PER_EXP + pos, -1)

        col_io = lax.broadcasted_iota(jnp.int32, (N_TOK, SLOTS), 1)
        c_my = (slot_rel == col_io).astype(jnp.float32)
        xg = jax.lax.dot_general(
            c_my, x_ref[:], (((0,), (0,)), ((), ())),
            preferred_element_type=jnp.float32)

        def qtr_ref(s, q):
            return comm_ref.at[pl.ds(s * SLOTS + q * QTR, QTR), :]

        def mk(src_slot, dst_slot, q, dev, sem_i, send_sems, recv_sems):
            return pltpu.make_async_remote_copy(
                src_ref=qtr_ref(src_slot, q), dst_ref=qtr_ref(dst_slot, q),
                send_sem=send_sems.at[sem_i], recv_sem=recv_sems.at[sem_i],
                device_id=(dev,), device_id_type=pl.DeviceIdType.MESH,
            )

        cw_sends = [[mk((N_DEV - r) % N_DEV, N_DEV - 1 - r, q, right,
                        NQ * r + q, cw_send_sems, cw_recv_sems)
                     for q in range(NQ)] for r in range(CW_ROUNDS)]
        cw_recvs = [[mk((N_DEV - r) % N_DEV, N_DEV - 1 - r, q, left,
                        NQ * r + q, cw_send_sems, cw_recv_sems)
                     for q in range(NQ)] for r in range(CW_ROUNDS)]
        ccw_sends = [[mk(r, r + 1, q, left,
                         NQ * r + q, ccw_send_sems, ccw_recv_sems)
                      for q in range(NQ)] for r in range(CCW_ROUNDS)]
        ccw_recvs = [[mk(r, r + 1, q, right,
                         NQ * r + q, ccw_send_sems, ccw_recv_sems)
                      for q in range(NQ)] for r in range(CCW_ROUNDS)]

        for le in range(E_LOC):
            sl = slice(le * SLOT_PER_EXP, (le + 1) * SLOT_PER_EXP)
            y_le = jnp.dot(xg[sl], w_ref[le],
                           preferred_element_type=jnp.float32)
            comm_ref[sl, :] = y_le.astype(jnp.bfloat16)
            cw_sends[0][le].start()
            ccw_sends[0][le].start()

        g_io = lax.broadcasted_iota(jnp.int32, (N_TOK, N_DEV * SLOTS), 1)
        c_all = (slot_rel == g_io).astype(jnp.bfloat16)

        for r in range(CW_ROUNDS):
            for q in range(NQ):
                cw_recvs[r][q].wait_recv()
                if r + 1 < CW_ROUNDS:
                    cw_sends[r + 1][q].start()
                if r < CCW_ROUNDS:
                    ccw_recvs[r][q].wait_recv()
                    if r + 1 < CCW_ROUNDS:
                        ccw_sends[r + 1][q].start()

        out_ref[:] = jnp.dot(c_all, comm_ref[:],
                             preferred_element_type=jnp.float32
                             ).astype(jnp.bfloat16)

        for pair in cw_sends + ccw_sends:
            for d in pair:
                d.wait_send()

    return pl.pallas_call(
        body,
        out_shape=jax.ShapeDtypeStruct((N_TOK, D_OUT), jnp.bfloat16),
        in_specs=[
            pl.BlockSpec(memory_space=pltpu.VMEM),
            pl.BlockSpec(memory_space=pltpu.VMEM),
            pl.BlockSpec(memory_space=pltpu.VMEM),
        ],
        out_specs=pl.BlockSpec(memory_space=pltpu.VMEM),
        scratch_shapes=[
            pltpu.VMEM((N_DEV * SLOTS, D_OUT), jnp.bfloat16),
            pltpu.SemaphoreType.DMA((NQ * CW_ROUNDS,)),
            pltpu.SemaphoreType.DMA((NQ * CW_ROUNDS,)),
            pltpu.SemaphoreType.DMA((NQ * CCW_ROUNDS,)),
            pltpu.SemaphoreType.DMA((NQ * CCW_ROUNDS,)),
        ],
        compiler_params=pltpu.CompilerParams(collective_id=0),
    )(x, route_idx, expert_W)
